# baseline (speedup 1.0000x reference)
import functools

import jax
import jax.numpy as jnp
from jax import lax
from jax.experimental import pallas as pl
from jax.experimental.pallas import tpu as pltpu
from jax.experimental.pallas import tpu_sc as plsc

_N = 10000
_E = 320000
_DH = 128
_DP = 48

_NC = 2
_NS = 16
_NW = _NC * _NS
_CH = 128
_K = 80
_K2 = _K + 8
_KH = _K // 2
_KS = _KH + 8
_SB = 1
_EPAD = _NW * _K * _CH
_NACC = 10112
_ZR = _NACC // _NS
_OR = _N // _NS

_R = 1000
_G = _N // _R


def _sc_degree(dst2d, ones, zeros):
    mesh = plsc.VectorSubcoreMesh(core_axis_name="c", subcore_axis_name="s")

    @functools.partial(
        pl.kernel,
        mesh=mesh,
        out_type=jax.ShapeDtypeStruct((_NC, _NACC, _DH), jnp.float32),
        scratch_types=[
            pltpu.VMEM((_K, _CH), jnp.int32),
            pltpu.VMEM((_CH, _DH), jnp.float32),
            pltpu.VMEM_SHARED((_NACC, _DH), jnp.float32),
        ],
    )
    def k(dst_hbm, ones_hbm, zeros_hbm, out_hbm, dst_v, ones_v, acc):
        c = lax.axis_index("c")
        s = lax.axis_index("s")
        wid = s * _NC + c
        pltpu.sync_copy(zeros_hbm, acc.at[pl.ds(s * _ZR, _ZR)])
        pltpu.sync_copy(dst_hbm.at[wid, pl.ds(0, _K)], dst_v)
        pltpu.sync_copy(ones_hbm, ones_v)
        plsc.subcore_barrier()

        def body(j, carry):
            pltpu.sync_copy(ones_v, acc.at[dst_v.at[j]], add=True)
            return carry

        lax.fori_loop(0, _K, body, 0)
        plsc.subcore_barrier()
        pltpu.sync_copy(acc.at[pl.ds(s * _ZR, _ZR)],
                        out_hbm.at[c, pl.ds(s * _ZR, _ZR)])

    return k(dst2d, ones, zeros)


def _sc_aggregate(hs, src2d, dst2d, zeros, d):
    mesh = plsc.VectorSubcoreMesh(core_axis_name="c", subcore_axis_name="s")

    @functools.partial(
        pl.kernel,
        mesh=mesh,
        out_type=jax.ShapeDtypeStruct((_NC, _NACC, d), jnp.float32),
        scratch_types=[
            pltpu.VMEM((_KS, _CH), jnp.int32),
            pltpu.VMEM((_KS, _CH), jnp.int32),
            pltpu.VMEM((_SB * _CH, d), jnp.float32),
            pltpu.VMEM_SHARED((_NACC, d), jnp.float32),
            pltpu.SemaphoreType.DMA,
        ],
    )
    def k(hs_hbm, src_hbm, dst_hbm, zeros_hbm, out_hbm,
          src_v, dst_v, rows_v, acc, sem):
        c = lax.axis_index("c")
        s = lax.axis_index("s")
        wid = s * _NC + c
        pltpu.sync_copy(zeros_hbm, acc.at[pl.ds(s * _ZR, _ZR)])
        plsc.subcore_barrier()

        for p in range(2):
            pltpu.sync_copy(src_hbm.at[wid, pl.ds(p * _KH, _KS)], src_v)
            pltpu.sync_copy(dst_hbm.at[wid, pl.ds(p * _KH, _KS)], dst_v)

            def body(j, carry):
                pltpu.async_copy(hs_hbm.at[src_v.at[j]], rows_v, sem).wait()
                pltpu.sync_copy(rows_v, acc.at[dst_v.at[j]], add=True)
                return carry

            lax.fori_loop(0, _KH, body, 0)
        plsc.subcore_barrier()
        pltpu.sync_copy(acc.at[pl.ds(s * _ZR, _ZR)],
                        out_hbm.at[c, pl.ds(s * _ZR, _ZR)])

    return k(hs, src2d, dst2d, zeros)


def _tc_scale_matmul(degp, x, w1):
    def body(degp_ref, x_ref, w_ref, dis_ref, hs_ref):
        degsum = degp_ref[0, :, :1] + degp_ref[1, :, :1]
        dis = lax.rsqrt(degsum + 1.0)
        dis_ref[...] = dis
        h = jnp.dot(x_ref[...], w_ref[...],
                    preferred_element_type=jnp.float32)
        hs_ref[...] = h * dis

    return pl.pallas_call(
        body,
        grid=(_G,),
        in_specs=[
            pl.BlockSpec((_NC, _R, _DH), lambda i: (0, i, 0)),
            pl.BlockSpec((_R, _DH), lambda i: (i, 0)),
            pl.BlockSpec((_DH, _DH), lambda i: (0, 0)),
        ],
        out_specs=[
            pl.BlockSpec((_R, 1), lambda i: (i, 0)),
            pl.BlockSpec((_R, _DH), lambda i: (i, 0)),
        ],
        out_shape=[
            jax.ShapeDtypeStruct((_N, 1), jnp.float32),
            jax.ShapeDtypeStruct((_N, _DH), jnp.float32),
        ],
    )(degp, x, w1)


def _tc_combine_stats(p1, hs1, dis, b1):
    def body(p_ref, hs_ref, dis_ref, b_ref, z_ref, st_ref):
        i = pl.program_id(0)
        z = (p_ref[0] + p_ref[1] + hs_ref[...]) * dis_ref[...] + b_ref[...]
        z_ref[...] = z
        st = jnp.concatenate(
            [jnp.sum(z, axis=0, keepdims=True),
             jnp.sum(z * z, axis=0, keepdims=True)], axis=0)

        @pl.when(i == 0)
        def _():
            st_ref[...] = st

        @pl.when(i != 0)
        def _():
            st_ref[...] = st_ref[...] + st

    return pl.pallas_call(
        body,
        grid=(_G,),
        in_specs=[
            pl.BlockSpec((_NC, _R, _DH), lambda i: (0, i, 0)),
            pl.BlockSpec((_R, _DH), lambda i: (i, 0)),
            pl.BlockSpec((_R, 1), lambda i: (i, 0)),
            pl.BlockSpec((1, _DH), lambda i: (0, 0)),
        ],
        out_specs=[
            pl.BlockSpec((_R, _DH), lambda i: (i, 0)),
            pl.BlockSpec((2, _DH), lambda i: (0, 0)),
        ],
        out_shape=[
            jax.ShapeDtypeStruct((_N, _DH), jnp.float32),
            jax.ShapeDtypeStruct((2, _DH), jnp.float32),
        ],
    )(p1, hs1, dis, b1)


def _tc_bn_relu(z, st, gamma, beta, dis):
    def body(z_ref, st_ref, g_ref, be_ref, dis_ref, zs_ref):
        st = st_ref[...]
        mean = st[0:1] * (1.0 / _N)
        var = st[1:2] * (1.0 / _N) - mean * mean
        zn = (z_ref[...] - mean) * lax.rsqrt(var + 1e-5)
        zr = jnp.maximum(zn * g_ref[...] + be_ref[...], 0.0)
        zs_ref[...] = zr * dis_ref[...]

    return pl.pallas_call(
        body,
        grid=(_G,),
        in_specs=[
            pl.BlockSpec((_R, _DH), lambda i: (i, 0)),
            pl.BlockSpec((2, _DH), lambda i: (0, 0)),
            pl.BlockSpec((1, _DH), lambda i: (0, 0)),
            pl.BlockSpec((1, _DH), lambda i: (0, 0)),
            pl.BlockSpec((_R, 1), lambda i: (i, 0)),
        ],
        out_specs=pl.BlockSpec((_R, _DH), lambda i: (i, 0)),
        out_shape=jax.ShapeDtypeStruct((_N, _DH), jnp.float32),
    )(z, st, gamma, beta, dis)


def _tc_final(p2, zs, dis, w2p, b2p):
    def body(p_ref, zs_ref, dis_ref, w_ref, b_ref, o_ref):
        t = (p_ref[0] + p_ref[1] + zs_ref[...]) * dis_ref[...]
        o_ref[...] = jnp.dot(t, w_ref[...],
                             preferred_element_type=jnp.float32) + b_ref[...]

    return pl.pallas_call(
        body,
        grid=(_G,),
        in_specs=[
            pl.BlockSpec((_NC, _R, _DH), lambda i: (0, i, 0)),
            pl.BlockSpec((_R, _DH), lambda i: (i, 0)),
            pl.BlockSpec((_R, 1), lambda i: (i, 0)),
            pl.BlockSpec((_DH, _DP), lambda i: (0, 0)),
            pl.BlockSpec((1, _DP), lambda i: (0, 0)),
        ],
        out_specs=pl.BlockSpec((_R, _DP), lambda i: (i, 0)),
        out_shape=jax.ShapeDtypeStruct((_N, _DP), jnp.float32),
    )(p2, zs, dis, w2p, b2p)


def kernel(x, adj_t, W1, b1, gamma1, beta1, W2, b2):
    src = adj_t[0].astype(jnp.int32)
    dst = adj_t[1].astype(jnp.int32)
    pad = _EPAD - _E
    src2d = jnp.concatenate(
        [jnp.concatenate([src, jnp.zeros((pad,), jnp.int32)])
         .reshape(_NW, _K, _CH),
         jnp.zeros((_NW, _K2 - _K, _CH), jnp.int32)], axis=1)
    dst2d = jnp.concatenate(
        [jnp.concatenate([dst, jnp.full((pad,), _N, jnp.int32)])
         .reshape(_NW, _K, _CH),
         jnp.full((_NW, _K2 - _K, _CH), _N, jnp.int32)], axis=1)

    ones128 = jnp.ones((_CH, _DH), jnp.float32)
    zeros128 = jnp.zeros((_ZR, _DH), jnp.float32)
    degp = _sc_degree(dst2d, ones128, zeros128)[:, :_N]

    dis, hs1 = _tc_scale_matmul(degp, x, W1)

    p1 = _sc_aggregate(hs1, src2d, dst2d, zeros128, _DH)[:, :_N]

    z, st = _tc_combine_stats(p1, hs1, dis, b1.reshape(1, _DH))

    zs = _tc_bn_relu(z, st, gamma1.reshape(1, _DH),
                     beta1.reshape(1, _DH), dis)

    p2 = _sc_aggregate(zs, src2d, dst2d, zeros128, _DH)[:, :_N]

    w2p = jnp.pad(W2, ((0, 0), (0, _DP - W2.shape[1])))
    b2p = jnp.pad(b2, (0, _DP - b2.shape[0])).reshape(1, _DP)
    out = _tc_final(p2, zs, dis, w2p, b2p)
    return out[:, :40]


# speedup vs baseline: 1.0018x; 1.0018x over previous
import functools

import jax
import jax.numpy as jnp
from jax import lax
from jax.experimental import pallas as pl
from jax.experimental.pallas import tpu as pltpu
from jax.experimental.pallas import tpu_sc as plsc

_N = 10000
_E = 320000
_DH = 128
_DP = 48

_NC = 2
_NS = 16
_NW = _NC * _NS
_CH = 128
_K = 80
_K2 = _K + 8
_KH = _K // 2
_KS = _KH + 8
_SB = 1
_EPAD = _NW * _K * _CH
_NACC = 10112
_ZR = _NACC // _NS
_OR = _N // _NS

_R = 1000
_G = _N // _R


def _sc_degree(dst2d, ones, zeros):
    mesh = plsc.VectorSubcoreMesh(core_axis_name="c", subcore_axis_name="s")

    @functools.partial(
        pl.kernel,
        mesh=mesh,
        out_type=jax.ShapeDtypeStruct((_NC, _NACC, _DH), jnp.float32),
        scratch_types=[
            pltpu.VMEM((_K, _CH), jnp.int32),
            pltpu.VMEM((_CH, _DH), jnp.float32),
            pltpu.VMEM_SHARED((_NACC, _DH), jnp.float32),
        ],
    )
    def k(dst_hbm, ones_hbm, zeros_hbm, out_hbm, dst_v, ones_v, acc):
        c = lax.axis_index("c")
        s = lax.axis_index("s")
        wid = s * _NC + c
        pltpu.sync_copy(zeros_hbm, acc.at[pl.ds(s * _ZR, _ZR)])
        pltpu.sync_copy(dst_hbm.at[wid, pl.ds(0, _K)], dst_v)
        pltpu.sync_copy(ones_hbm, ones_v)
        plsc.subcore_barrier()

        def body(j, carry):
            pltpu.sync_copy(ones_v, acc.at[dst_v.at[j]], add=True)
            return carry

        lax.fori_loop(0, _K, body, 0)
        plsc.subcore_barrier()
        pltpu.sync_copy(acc.at[pl.ds(s * _ZR, _ZR)],
                        out_hbm.at[c, pl.ds(s * _ZR, _ZR)])

    return k(dst2d, ones, zeros)


def _sc_aggregate(hs, src2d, dst2d, zeros, d):
    mesh = plsc.VectorSubcoreMesh(core_axis_name="c", subcore_axis_name="s")

    @functools.partial(
        pl.kernel,
        mesh=mesh,
        out_type=jax.ShapeDtypeStruct((_NC, _NACC, d), jnp.float32),
        scratch_types=[
            pltpu.VMEM((_K2, _CH), jnp.int32),
            pltpu.VMEM((_K2, _CH), jnp.int32),
            pltpu.VMEM((_SB * _CH, d), jnp.float32),
            pltpu.VMEM_SHARED((_NACC, d), jnp.float32),
            pltpu.SemaphoreType.DMA,
        ],
    )
    def k(hs_hbm, src_hbm, dst_hbm, zeros_hbm, out_hbm,
          src_v, dst_v, rows_v, acc, sem):
        c = lax.axis_index("c")
        s = lax.axis_index("s")
        wid = s * _NC + c
        pltpu.sync_copy(zeros_hbm, acc.at[pl.ds(s * _ZR, _ZR)])
        pltpu.sync_copy(src_hbm.at[wid], src_v)
        pltpu.sync_copy(dst_hbm.at[wid], dst_v)
        plsc.subcore_barrier()

        def body(j, carry):
            pltpu.async_copy(hs_hbm.at[src_v.at[j]], rows_v, sem).wait()
            pltpu.sync_copy(rows_v, acc.at[dst_v.at[j]], add=True)
            return carry

        lax.fori_loop(0, _K, body, 0)
        plsc.subcore_barrier()
        pltpu.sync_copy(acc.at[pl.ds(s * _ZR, _ZR)],
                        out_hbm.at[c, pl.ds(s * _ZR, _ZR)])

    return k(hs, src2d, dst2d, zeros)


def _tc_scale_matmul(degp, x, w1):
    def body(degp_ref, x_ref, w_ref, dis_ref, hs_ref):
        degsum = degp_ref[0, :, :1] + degp_ref[1, :, :1]
        dis = lax.rsqrt(degsum + 1.0)
        dis_ref[...] = dis
        h = jnp.dot(x_ref[...], w_ref[...],
                    preferred_element_type=jnp.float32)
        hs_ref[...] = h * dis

    return pl.pallas_call(
        body,
        grid=(_G,),
        in_specs=[
            pl.BlockSpec((_NC, _R, _DH), lambda i: (0, i, 0)),
            pl.BlockSpec((_R, _DH), lambda i: (i, 0)),
            pl.BlockSpec((_DH, _DH), lambda i: (0, 0)),
        ],
        out_specs=[
            pl.BlockSpec((_R, 1), lambda i: (i, 0)),
            pl.BlockSpec((_R, _DH), lambda i: (i, 0)),
        ],
        out_shape=[
            jax.ShapeDtypeStruct((_N, 1), jnp.float32),
            jax.ShapeDtypeStruct((_N, _DH), jnp.float32),
        ],
    )(degp, x, w1)


def _tc_combine_stats(p1, hs1, dis, b1):
    def body(p_ref, hs_ref, dis_ref, b_ref, z_ref, st_ref):
        i = pl.program_id(0)
        z = (p_ref[0] + p_ref[1] + hs_ref[...]) * dis_ref[...] + b_ref[...]
        z_ref[...] = z
        st = jnp.concatenate(
            [jnp.sum(z, axis=0, keepdims=True),
             jnp.sum(z * z, axis=0, keepdims=True)], axis=0)

        @pl.when(i == 0)
        def _():
            st_ref[...] = st

        @pl.when(i != 0)
        def _():
            st_ref[...] = st_ref[...] + st

    return pl.pallas_call(
        body,
        grid=(_G,),
        in_specs=[
            pl.BlockSpec((_NC, _R, _DH), lambda i: (0, i, 0)),
            pl.BlockSpec((_R, _DH), lambda i: (i, 0)),
            pl.BlockSpec((_R, 1), lambda i: (i, 0)),
            pl.BlockSpec((1, _DH), lambda i: (0, 0)),
        ],
        out_specs=[
            pl.BlockSpec((_R, _DH), lambda i: (i, 0)),
            pl.BlockSpec((2, _DH), lambda i: (0, 0)),
        ],
        out_shape=[
            jax.ShapeDtypeStruct((_N, _DH), jnp.float32),
            jax.ShapeDtypeStruct((2, _DH), jnp.float32),
        ],
    )(p1, hs1, dis, b1)


def _tc_bn_relu(z, st, gamma, beta, dis):
    def body(z_ref, st_ref, g_ref, be_ref, dis_ref, zs_ref):
        st = st_ref[...]
        mean = st[0:1] * (1.0 / _N)
        var = st[1:2] * (1.0 / _N) - mean * mean
        zn = (z_ref[...] - mean) * lax.rsqrt(var + 1e-5)
        zr = jnp.maximum(zn * g_ref[...] + be_ref[...], 0.0)
        zs_ref[...] = zr * dis_ref[...]

    return pl.pallas_call(
        body,
        grid=(_G,),
        in_specs=[
            pl.BlockSpec((_R, _DH), lambda i: (i, 0)),
            pl.BlockSpec((2, _DH), lambda i: (0, 0)),
            pl.BlockSpec((1, _DH), lambda i: (0, 0)),
            pl.BlockSpec((1, _DH), lambda i: (0, 0)),
            pl.BlockSpec((_R, 1), lambda i: (i, 0)),
        ],
        out_specs=pl.BlockSpec((_R, _DH), lambda i: (i, 0)),
        out_shape=jax.ShapeDtypeStruct((_N, _DH), jnp.float32),
    )(z, st, gamma, beta, dis)


def _tc_final(p2, zs, dis, w2p, b2p):
    def body(p_ref, zs_ref, dis_ref, w_ref, b_ref, o_ref):
        t = (p_ref[0] + p_ref[1] + zs_ref[...]) * dis_ref[...]
        o_ref[...] = jnp.dot(t, w_ref[...],
                             preferred_element_type=jnp.float32) + b_ref[...]

    return pl.pallas_call(
        body,
        grid=(_G,),
        in_specs=[
            pl.BlockSpec((_NC, _R, _DH), lambda i: (0, i, 0)),
            pl.BlockSpec((_R, _DH), lambda i: (i, 0)),
            pl.BlockSpec((_R, 1), lambda i: (i, 0)),
            pl.BlockSpec((_DH, _DP), lambda i: (0, 0)),
            pl.BlockSpec((1, _DP), lambda i: (0, 0)),
        ],
        out_specs=pl.BlockSpec((_R, _DP), lambda i: (i, 0)),
        out_shape=jax.ShapeDtypeStruct((_N, _DP), jnp.float32),
    )(p2, zs, dis, w2p, b2p)


def kernel(x, adj_t, W1, b1, gamma1, beta1, W2, b2):
    src = adj_t[0].astype(jnp.int32)
    dst = adj_t[1].astype(jnp.int32)
    pad = _EPAD - _E
    src2d = jnp.concatenate(
        [jnp.concatenate([src, jnp.zeros((pad,), jnp.int32)])
         .reshape(_NW, _K, _CH),
         jnp.zeros((_NW, _K2 - _K, _CH), jnp.int32)], axis=1)
    dst2d = jnp.concatenate(
        [jnp.concatenate([dst, jnp.full((pad,), _N, jnp.int32)])
         .reshape(_NW, _K, _CH),
         jnp.full((_NW, _K2 - _K, _CH), _N, jnp.int32)], axis=1)

    ones128 = jnp.ones((_CH, _DH), jnp.float32)
    zeros128 = jnp.zeros((_ZR, _DH), jnp.float32)
    degp = _sc_degree(dst2d, ones128, zeros128)[:, :_N]

    dis, hs1 = _tc_scale_matmul(degp, x, W1)

    p1 = _sc_aggregate(hs1, src2d, dst2d, zeros128, _DH)[:, :_N]

    z, st = _tc_combine_stats(p1, hs1, dis, b1.reshape(1, _DH))

    zs = _tc_bn_relu(z, st, gamma1.reshape(1, _DH),
                     beta1.reshape(1, _DH), dis)

    p2 = _sc_aggregate(zs, src2d, dst2d, zeros128, _DH)[:, :_N]

    w2p = jnp.pad(W2, ((0, 0), (0, _DP - W2.shape[1])))
    b2p = jnp.pad(b2, (0, _DP - b2.shape[0])).reshape(1, _DP)
    out = _tc_final(p2, zs, dis, w2p, b2p)
    return out[:, :40]


# speedup vs baseline: 1.5308x; 1.5280x over previous
import functools

import jax
import jax.numpy as jnp
from jax import lax
from jax.experimental import pallas as pl
from jax.experimental.pallas import tpu as pltpu
from jax.experimental.pallas import tpu_sc as plsc

_N = 10000
_E = 320000
_DH = 128
_DP = 48

_NC = 2
_NS = 16
_NW = _NC * _NS
_CH = 128
_K = 79
_K2 = _K
_SB = 1
_EPAD = _NW * _K * _CH
_NACC = 10240
_ZR = _NACC // _NS
_OR = _N // _NS

_R = 1000
_G = _N // _R


def _sc_degree(dst2d, ones, zeros):
    mesh = plsc.VectorSubcoreMesh(core_axis_name="c", subcore_axis_name="s")

    @functools.partial(
        pl.kernel,
        mesh=mesh,
        out_type=jax.ShapeDtypeStruct((_NC, _NACC, _DH), jnp.float32),
        scratch_types=[
            pltpu.VMEM((_K, _CH), jnp.int32),
            pltpu.VMEM((_CH, _DH), jnp.float32),
            pltpu.VMEM_SHARED((_NACC, _DH), jnp.float32),
        ],
    )
    def k(dst_hbm, ones_hbm, zeros_hbm, out_hbm, dst_v, ones_v, acc):
        c = lax.axis_index("c")
        s = lax.axis_index("s")
        wid = s * _NC + c
        pltpu.sync_copy(zeros_hbm, acc.at[pl.ds(s * _ZR, _ZR)])
        pltpu.sync_copy(dst_hbm.at[wid], dst_v)
        pltpu.sync_copy(ones_hbm, ones_v)
        plsc.subcore_barrier()

        def body(j, carry):
            pltpu.sync_copy(ones_v, acc.at[dst_v.at[j]], add=True)
            return carry

        lax.fori_loop(0, _K, body, 0)
        plsc.subcore_barrier()
        pltpu.sync_copy(acc.at[pl.ds(s * _ZR, _ZR)],
                        out_hbm.at[c, pl.ds(s * _ZR, _ZR)])

    return k(dst2d, ones, zeros)


def _sc_aggregate(hs, src2d, dst2d, zeros, d):
    mesh = plsc.VectorSubcoreMesh(core_axis_name="c", subcore_axis_name="s")

    @functools.partial(
        pl.kernel,
        mesh=mesh,
        out_type=jax.ShapeDtypeStruct((_NC, _NACC, d), jnp.float32),
        scratch_types=[
            pltpu.VMEM((_K2, _CH), jnp.int32),
            pltpu.VMEM((_K2, _CH), jnp.int32),
            pltpu.VMEM((_SB * _CH, d), jnp.float32),
            pltpu.VMEM_SHARED((_NACC, d), jnp.float32),
            pltpu.SemaphoreType.DMA,
        ],
    )
    def k(hs_hbm, src_hbm, dst_hbm, zeros_hbm, out_hbm,
          src_v, dst_v, rows_v, acc, sem):
        c = lax.axis_index("c")
        s = lax.axis_index("s")
        wid = s * _NC + c
        pltpu.sync_copy(zeros_hbm, acc.at[pl.ds(s * _ZR, _ZR)])
        pltpu.sync_copy(src_hbm.at[wid], src_v)
        pltpu.sync_copy(dst_hbm.at[wid], dst_v)
        plsc.subcore_barrier()

        def body(j, carry):
            pltpu.async_copy(hs_hbm.at[src_v.at[j]], rows_v, sem).wait()
            pltpu.sync_copy(rows_v, acc.at[dst_v.at[j]], add=True)
            return carry

        lax.fori_loop(0, _K, body, 0)
        plsc.subcore_barrier()
        pltpu.sync_copy(acc.at[pl.ds(s * _ZR, _ZR)],
                        out_hbm.at[c, pl.ds(s * _ZR, _ZR)])

    return k(hs, src2d, dst2d, zeros)


def _tc_scale_matmul(degp, x, w1):
    def body(degp_ref, x_ref, w_ref, dis_ref, hs_ref):
        degsum = degp_ref[0, :, :1] + degp_ref[1, :, :1]
        dis = lax.rsqrt(degsum + 1.0)
        dis_ref[...] = dis
        h = jnp.dot(x_ref[...], w_ref[...],
                    preferred_element_type=jnp.float32)
        hs_ref[...] = h * dis

    return pl.pallas_call(
        body,
        grid=(_G,),
        in_specs=[
            pl.BlockSpec((_NC, _R, _DH), lambda i: (0, i, 0)),
            pl.BlockSpec((_R, _DH), lambda i: (i, 0)),
            pl.BlockSpec((_DH, _DH), lambda i: (0, 0)),
        ],
        out_specs=[
            pl.BlockSpec((_R, 1), lambda i: (i, 0)),
            pl.BlockSpec((_R, _DH), lambda i: (i, 0)),
        ],
        out_shape=[
            jax.ShapeDtypeStruct((_N, 1), jnp.float32),
            jax.ShapeDtypeStruct((_N, _DH), jnp.float32),
        ],
    )(degp, x, w1)


def _tc_combine_stats(p1, hs1, dis, b1):
    def body(p_ref, hs_ref, dis_ref, b_ref, z_ref, st_ref):
        i = pl.program_id(0)
        z = (p_ref[0] + p_ref[1] + hs_ref[...]) * dis_ref[...] + b_ref[...]
        z_ref[...] = z
        st = jnp.concatenate(
            [jnp.sum(z, axis=0, keepdims=True),
             jnp.sum(z * z, axis=0, keepdims=True)], axis=0)

        @pl.when(i == 0)
        def _():
            st_ref[...] = st

        @pl.when(i != 0)
        def _():
            st_ref[...] = st_ref[...] + st

    return pl.pallas_call(
        body,
        grid=(_G,),
        in_specs=[
            pl.BlockSpec((_NC, _R, _DH), lambda i: (0, i, 0)),
            pl.BlockSpec((_R, _DH), lambda i: (i, 0)),
            pl.BlockSpec((_R, 1), lambda i: (i, 0)),
            pl.BlockSpec((1, _DH), lambda i: (0, 0)),
        ],
        out_specs=[
            pl.BlockSpec((_R, _DH), lambda i: (i, 0)),
            pl.BlockSpec((2, _DH), lambda i: (0, 0)),
        ],
        out_shape=[
            jax.ShapeDtypeStruct((_N, _DH), jnp.float32),
            jax.ShapeDtypeStruct((2, _DH), jnp.float32),
        ],
    )(p1, hs1, dis, b1)


def _tc_bn_relu(z, st, gamma, beta, dis):
    def body(z_ref, st_ref, g_ref, be_ref, dis_ref, zs_ref):
        st = st_ref[...]
        mean = st[0:1] * (1.0 / _N)
        var = st[1:2] * (1.0 / _N) - mean * mean
        zn = (z_ref[...] - mean) * lax.rsqrt(var + 1e-5)
        zr = jnp.maximum(zn * g_ref[...] + be_ref[...], 0.0)
        zs_ref[...] = zr * dis_ref[...]

    return pl.pallas_call(
        body,
        grid=(_G,),
        in_specs=[
            pl.BlockSpec((_R, _DH), lambda i: (i, 0)),
            pl.BlockSpec((2, _DH), lambda i: (0, 0)),
            pl.BlockSpec((1, _DH), lambda i: (0, 0)),
            pl.BlockSpec((1, _DH), lambda i: (0, 0)),
            pl.BlockSpec((_R, 1), lambda i: (i, 0)),
        ],
        out_specs=pl.BlockSpec((_R, _DH), lambda i: (i, 0)),
        out_shape=jax.ShapeDtypeStruct((_N, _DH), jnp.float32),
    )(z, st, gamma, beta, dis)


def _tc_final(p2, zs, dis, w2p, b2p):
    def body(p_ref, zs_ref, dis_ref, w_ref, b_ref, o_ref):
        t = (p_ref[0] + p_ref[1] + zs_ref[...]) * dis_ref[...]
        o_ref[...] = jnp.dot(t, w_ref[...],
                             preferred_element_type=jnp.float32) + b_ref[...]

    return pl.pallas_call(
        body,
        grid=(_G,),
        in_specs=[
            pl.BlockSpec((_NC, _R, _DH), lambda i: (0, i, 0)),
            pl.BlockSpec((_R, _DH), lambda i: (i, 0)),
            pl.BlockSpec((_R, 1), lambda i: (i, 0)),
            pl.BlockSpec((_DH, _DP), lambda i: (0, 0)),
            pl.BlockSpec((1, _DP), lambda i: (0, 0)),
        ],
        out_specs=pl.BlockSpec((_R, _DP), lambda i: (i, 0)),
        out_shape=jax.ShapeDtypeStruct((_N, _DP), jnp.float32),
    )(p2, zs, dis, w2p, b2p)


def kernel(x, adj_t, W1, b1, gamma1, beta1, W2, b2):
    src = adj_t[0].astype(jnp.int32)
    dst = adj_t[1].astype(jnp.int32)
    pad = _EPAD - _E
    src2d = jnp.concatenate(
        [src, jnp.zeros((pad,), jnp.int32)]).reshape(_NW, _K, _CH)
    dst2d = jnp.concatenate(
        [dst, jnp.full((pad,), _N, jnp.int32)]).reshape(_NW, _K, _CH)

    ones128 = jnp.ones((_CH, _DH), jnp.float32)
    zeros128 = jnp.zeros((_ZR, _DH), jnp.float32)
    degp = _sc_degree(dst2d, ones128, zeros128)[:, :_N]

    dis, hs1 = _tc_scale_matmul(degp, x, W1)

    p1 = _sc_aggregate(hs1, src2d, dst2d, zeros128, _DH)[:, :_N]

    z, st = _tc_combine_stats(p1, hs1, dis, b1.reshape(1, _DH))

    zs = _tc_bn_relu(z, st, gamma1.reshape(1, _DH),
                     beta1.reshape(1, _DH), dis)

    p2 = _sc_aggregate(zs, src2d, dst2d, zeros128, _DH)[:, :_N]

    w2p = jnp.pad(W2, ((0, 0), (0, _DP - W2.shape[1])))
    b2p = jnp.pad(b2, (0, _DP - b2.shape[0])).reshape(1, _DP)
    out = _tc_final(p2, zs, dis, w2p, b2p)
    return out[:, :40]
